# baseline (speedup 1.0000x reference)
import functools

import jax
import jax.numpy as jnp
from jax import lax
from jax.experimental import pallas as pl
from jax.experimental.pallas import tpu as pltpu
from jax.experimental.pallas import tpu_sc as plsc

S, T = 4096, 200
B = S * T
D = 64
NC, NS = 2, 16
NW = NC * NS
BPW = B // NW
C = 128
NCHUNK = BPW // C

_vector_mesh = plsc.VectorSubcoreMesh(
    core_axis_name="core", subcore_axis_name="subcore"
)


@jax.jit
def _gather_sc(table, indices):
    @functools.partial(
        pl.kernel,
        out_type=jax.ShapeDtypeStruct((B, D), jnp.float32),
        mesh=_vector_mesh,
        scratch_types=[
            pltpu.VMEM((C,), jnp.int32),
            pltpu.VMEM((C, D), jnp.float32),
            pltpu.SemaphoreType.DMA,
        ],
        compiler_params=pltpu.CompilerParams(use_tc_tiling_on_sc=False),
    )
    def kern(tab_hbm, idx_hbm, out_hbm, idx_v, rows_v, sem):
        wid = lax.axis_index("subcore") * NC + lax.axis_index("core")
        base = wid * BPW

        @pl.loop(0, NCHUNK)
        def _(i):
            off = base + i * C
            pltpu.sync_copy(idx_hbm.at[pl.ds(off, C)], idx_v)
            pltpu.async_copy(tab_hbm.at[idx_v], rows_v, sem).wait()
            pltpu.sync_copy(rows_v, out_hbm.at[pl.ds(off, C)])

    return kern(table, indices)


def kernel(tokenized_sentence, table):
    idx = tokenized_sentence.reshape(B)
    out = _gather_sc(table, idx)
    return out.reshape(S, T, D)


# speedup vs baseline: 1.1921x; 1.1921x over previous
import functools

import jax
import jax.numpy as jnp
from jax import lax
from jax.experimental import pallas as pl
from jax.experimental.pallas import tpu as pltpu
from jax.experimental.pallas import tpu_sc as plsc

S, T = 4096, 200
B = S * T
D = 64
NC, NS = 2, 16
NW = NC * NS
BPW = B // NW
C = 512
NCHUNK = BPW // C
NBUF = 2

_vector_mesh = plsc.VectorSubcoreMesh(
    core_axis_name="core", subcore_axis_name="subcore"
)


@jax.jit
def _gather_sc(table, indices):
    @functools.partial(
        pl.kernel,
        out_type=jax.ShapeDtypeStruct((B, D), jnp.float32),
        mesh=_vector_mesh,
        scratch_types=[
            pltpu.VMEM((NBUF, C), jnp.int32),
            pltpu.VMEM((NBUF, C, D), jnp.float32),
            pltpu.SemaphoreType.DMA((NBUF,)),
            pltpu.SemaphoreType.DMA((NBUF,)),
            pltpu.SemaphoreType.DMA((NBUF,)),
        ],
        compiler_params=pltpu.CompilerParams(use_tc_tiling_on_sc=False),
    )
    def kern(tab_hbm, idx_hbm, out_hbm, idx_v, rows_v, isem, gsem, osem):
        wid = lax.axis_index("subcore") * NC + lax.axis_index("core")
        base = wid * BPW

        for b in range(NBUF):
            pltpu.async_copy(
                idx_hbm.at[pl.ds(base + b * C, C)], idx_v.at[b], isem.at[b]
            )

        @pl.loop(0, NCHUNK, step=NBUF)
        def _(i):
            for b in range(NBUF):
                off = base + (i + b) * C

                @pl.when(i > 0)
                def _():
                    pltpu.make_async_copy(
                        rows_v.at[b], out_hbm.at[pl.ds(off, C)], osem.at[b]
                    ).wait()

                pltpu.make_async_copy(
                    idx_hbm.at[pl.ds(off, C)], idx_v.at[b], isem.at[b]
                ).wait()

                pltpu.async_copy(
                    tab_hbm.at[idx_v.at[b]], rows_v.at[b], gsem.at[b]
                ).wait()

                @pl.when(i + NBUF < NCHUNK)
                def _():
                    pltpu.async_copy(
                        idx_hbm.at[pl.ds(off + NBUF * C, C)],
                        idx_v.at[b],
                        isem.at[b],
                    )

                pltpu.async_copy(
                    rows_v.at[b], out_hbm.at[pl.ds(off, C)], osem.at[b]
                )

        for b in range(NBUF):
            pltpu.make_async_copy(
                rows_v.at[b], out_hbm.at[pl.ds(base, C)], osem.at[b]
            ).wait()

    return kern(table, indices)


def kernel(tokenized_sentence, table):
    idx = tokenized_sentence.reshape(B)
    out = _gather_sc(table, idx)
    return out.reshape(S, T, D)
